# baseline (speedup 1.0000x reference)
import functools
import jax
import jax.numpy as jnp
from jax import lax
from jax.experimental import pallas as pl
from jax.experimental.pallas import tpu as pltpu, tpu_sc as plsc

_BASE_VOCAB_SHIFT = 100
_COORD_VOCAB_SHIFT = 1000
_QUANT_BINS = 1000
_MAX_INPUT_SIZE = 1024.0

_B, _S, _V = 32, 500, 3000
_N = _S // 5
_BT = 16
_BS = _B - _BT
_HALF = 50
_HP = 56
_NP = 2 * _HP
_W = 8

_NC = 2


@functools.cache
def _sc_gather_rows_fn():
    mesh = plsc.VectorSubcoreMesh(core_axis_name="c", subcore_axis_name="s")

    @functools.partial(
        pl.kernel, mesh=mesh,
        out_type=jax.ShapeDtypeStruct((_BS * _NP, _V), jnp.float32),
        scratch_types=[
            pltpu.VMEM((_W, _V), jnp.float32),
            pltpu.VMEM((_W, _V), jnp.float32),
            pltpu.SemaphoreType.DMA,
            pltpu.SemaphoreType.DMA,
            pltpu.SemaphoreType.DMA,
            pltpu.SemaphoreType.DMA,
        ],
    )
    def _sc_gather_rows(x_hbm, out_hbm, rows_a, rows_b,
                        gsem_a, gsem_b, wsem_a, wsem_b):
        w = lax.axis_index("s") * _NC + lax.axis_index("c")
        il = w // 2
        half = w % 2
        img = il + _BT
        dst0 = il * _NP + half * _HP
        row0 = half * _HALF
        bufs = (rows_a, rows_b)
        gsems = (gsem_a, gsem_b)
        wsems = (wsem_a, wsem_b)
        nwave = -(-_HALF // _W)

        def start_gathers(g, buf, sem):
            base = g * _W
            return [
                pltpu.async_copy(
                    x_hbm.at[img, 5 * (row0 + base + k) + 4],
                    buf.at[k], sem)
                for k in range(min(_W, _HALF - base))
            ]

        def start_write(g, buf, sem):
            dst = out_hbm.at[pl.ds(dst0 + _W * g, _W)]
            return [pltpu.async_copy(buf, dst, sem)]

        pend_g = start_gathers(0, bufs[0], gsems[0])
        pend_w = []
        for g in range(nwave):
            cur = g % 2
            for h in pend_g:
                h.wait()
            for h in pend_w:
                h.wait()
            if g + 1 < nwave:
                pend_g = start_gathers(g + 1, bufs[1 - cur], gsems[1 - cur])
            pend_w = start_write(g, bufs[cur], wsems[cur])
        for h in pend_w:
            h.wait()

    return _sc_gather_rows


def _core_decode(x, col):
    m = jnp.max(x, axis=-1, keepdims=True)
    denom = jnp.sum(jnp.exp(x - m), axis=-1, keepdims=True)
    inwin = (col >= _BASE_VOCAB_SHIFT) & (col < _COORD_VOCAB_SHIFT)
    xm = jnp.where(inwin, x, -jnp.inf)
    mw = jnp.max(xm, axis=-1, keepdims=True)
    idx = jnp.argmax(xm, axis=-1, keepdims=True)
    cls = jnp.maximum(idx - _BASE_VOCAB_SHIFT, 0)
    score = jnp.exp(mw - m) / denom
    return cls, score


def _bbox_decode(seq_ref, orig_ref, size_ref, k):
    sq = seq_ref[k]
    q = (sq - _COORD_VOCAB_SHIFT).astype(jnp.float32) / (_QUANT_BINS - 1)
    q = jnp.clip(q, 0.0, 1.0)
    sc = (_MAX_INPUT_SIZE / size_ref[k]) * orig_ref[k]
    bbox = jnp.concatenate(
        [q[:, 1:2], q[:, 0:1], q[:, 3:4], q[:, 2:3]], axis=1)
    scl4 = jnp.concatenate(
        [sc[:, 0:1], sc[:, 1:2], sc[:, 0:1], sc[:, 1:2]], axis=1)
    return bbox * scl4


def _tc_full_body(*refs):
    logits_a, logits_b, seq_ref, orig_ref, size_ref = refs[:5]
    cls_ref, bbox_ref, score_ref = refs[5:]
    for k, lref in enumerate((logits_a, logits_b)):
        x_all = lref[0]
        s, v = x_all.shape
        n = s // 5
        x = jnp.concatenate(
            [x_all[5 * i + 4:5 * i + 5] for i in range(n)])
        col = lax.broadcasted_iota(jnp.int32, (1, v), 1)
        cls, score = _core_decode(x, col)
        cls_ref[k] = cls
        score_ref[k] = score
        bbox_ref[k] = _bbox_decode(seq_ref, orig_ref, size_ref, k)


def _tc_compact_body(x_ref, seq_ref, orig_ref, size_ref,
                     cls_ref, bbox_ref, score_ref):
    x_np = x_ref[...]
    rows, v = x_np.shape
    x = jnp.concatenate([x_np[:_HALF], x_np[_HP:_HP + _HALF]])
    col = lax.broadcasted_iota(jnp.int32, (1, v), 1)
    cls, score = _core_decode(x, col)
    cls_ref[0] = cls
    score_ref[0] = score
    bbox_ref[0] = _bbox_decode(seq_ref, orig_ref, size_ref, 0)


def kernel(pred_seq_logits, pred_seq, orig_size, size, image_id):
    b, s, v = pred_seq_logits.shape
    n = s // 5
    seq3 = pred_seq.reshape(b, n, 5)
    orig_f = orig_size.astype(jnp.float32).reshape(b, 1, 2)
    size_f = size.astype(jnp.float32).reshape(b, 1, 2)

    compact = _sc_gather_rows_fn()(pred_seq_logits)

    def logits_spec(k):
        return pl.BlockSpec((1, s, v), lambda i, k=k: (2 * i + k, 0, 0))

    cls1, bbox1, score1 = pl.pallas_call(
        _tc_full_body,
        grid=(_BT // 2,),
        in_specs=[logits_spec(0), logits_spec(1)] + [
            pl.BlockSpec((2, n, 5), lambda i: (i, 0, 0)),
            pl.BlockSpec((2, 1, 2), lambda i: (i, 0, 0)),
            pl.BlockSpec((2, 1, 2), lambda i: (i, 0, 0)),
        ],
        out_specs=[
            pl.BlockSpec((2, n, 1), lambda i: (i, 0, 0)),
            pl.BlockSpec((2, n, 4), lambda i: (i, 0, 0)),
            pl.BlockSpec((2, n, 1), lambda i: (i, 0, 0)),
        ],
        out_shape=[
            jax.ShapeDtypeStruct((_BT, n, 1), jnp.int32),
            jax.ShapeDtypeStruct((_BT, n, 4), jnp.float32),
            jax.ShapeDtypeStruct((_BT, n, 1), jnp.float32),
        ],
        compiler_params=pltpu.CompilerParams(
            dimension_semantics=("arbitrary",)),
    )(pred_seq_logits, pred_seq_logits, seq3, orig_f, size_f)

    cls2, bbox2, score2 = pl.pallas_call(
        _tc_compact_body,
        grid=(_BS,),
        in_specs=[
            pl.BlockSpec((_NP, v), lambda i: (i, 0)),
            pl.BlockSpec((1, n, 5), lambda i: (i + _BT, 0, 0)),
            pl.BlockSpec((1, 1, 2), lambda i: (i + _BT, 0, 0)),
            pl.BlockSpec((1, 1, 2), lambda i: (i + _BT, 0, 0)),
        ],
        out_specs=[
            pl.BlockSpec((1, n, 1), lambda i: (i, 0, 0)),
            pl.BlockSpec((1, n, 4), lambda i: (i, 0, 0)),
            pl.BlockSpec((1, n, 1), lambda i: (i, 0, 0)),
        ],
        out_shape=[
            jax.ShapeDtypeStruct((_BS, n, 1), jnp.int32),
            jax.ShapeDtypeStruct((_BS, n, 4), jnp.float32),
            jax.ShapeDtypeStruct((_BS, n, 1), jnp.float32),
        ],
        compiler_params=pltpu.CompilerParams(
            dimension_semantics=("arbitrary",)),
    )(compact, seq3, orig_f, size_f)

    cls = jnp.concatenate([cls1[..., 0], cls2[..., 0]])
    bbox = jnp.concatenate([bbox1, bbox2])
    score = jnp.concatenate([score1[..., 0], score2[..., 0]])
    return cls, bbox, score


# speedup vs baseline: 1.0765x; 1.0765x over previous
import jax
import jax.numpy as jnp
from jax import lax
from jax.experimental import pallas as pl
from jax.experimental.pallas import tpu as pltpu

_BASE_VOCAB_SHIFT = 100
_COORD_VOCAB_SHIFT = 1000
_QUANT_BINS = 1000
_MAX_INPUT_SIZE = 1024.0
_K = 4


def _decode_one(x_all, k, seq_ref, orig_ref, size_ref,
                cls_ref, bbox_ref, score_ref):
    s, v = x_all.shape
    n = s // 5
    x = jnp.concatenate([x_all[5 * i + 4:5 * i + 5] for i in range(n)])
    m = jnp.max(x, axis=-1, keepdims=True)
    denom = jnp.sum(jnp.exp(x - m), axis=-1, keepdims=True)
    col = lax.broadcasted_iota(jnp.int32, (1, v), 1)
    inwin = (col >= _BASE_VOCAB_SHIFT) & (col < _COORD_VOCAB_SHIFT)
    xm = jnp.where(inwin, x, -jnp.inf)
    mw = jnp.max(xm, axis=-1, keepdims=True)
    idx = jnp.argmax(xm, axis=-1, keepdims=True)
    cls_ref[k] = jnp.maximum(idx - _BASE_VOCAB_SHIFT, 0)
    score_ref[k] = jnp.exp(mw - m) / denom

    sq = seq_ref[k]
    q = (sq - _COORD_VOCAB_SHIFT).astype(jnp.float32) / (_QUANT_BINS - 1)
    q = jnp.clip(q, 0.0, 1.0)
    sc = (_MAX_INPUT_SIZE / size_ref[k]) * orig_ref[k]
    bbox = jnp.concatenate(
        [q[:, 1:2], q[:, 0:1], q[:, 3:4], q[:, 2:3]], axis=1)
    scl4 = jnp.concatenate(
        [sc[:, 0:1], sc[:, 1:2], sc[:, 0:1], sc[:, 1:2]], axis=1)
    bbox_ref[k] = bbox * scl4


def _decode_body(*refs):
    logits_refs = refs[:_K]
    seq_ref, orig_ref, size_ref, cls_ref, bbox_ref, score_ref = refs[_K:]
    for k in range(_K):
        _decode_one(logits_refs[k][0], k, seq_ref, orig_ref, size_ref,
                    cls_ref, bbox_ref, score_ref)


def kernel(pred_seq_logits, pred_seq, orig_size, size, image_id):
    b, s, v = pred_seq_logits.shape
    n = s // 5
    seq3 = pred_seq.reshape(b, n, 5)
    orig_f = orig_size.astype(jnp.float32).reshape(b, 1, 2)
    size_f = size.astype(jnp.float32).reshape(b, 1, 2)

    def logits_spec(k):
        return pl.BlockSpec((1, s, v), lambda i, k=k: (_K * i + k, 0, 0))

    cls, bbox, score = pl.pallas_call(
        _decode_body,
        grid=(b // _K,),
        in_specs=[logits_spec(k) for k in range(_K)] + [
            pl.BlockSpec((_K, n, 5), lambda i: (i, 0, 0)),
            pl.BlockSpec((_K, 1, 2), lambda i: (i, 0, 0)),
            pl.BlockSpec((_K, 1, 2), lambda i: (i, 0, 0)),
        ],
        out_specs=[
            pl.BlockSpec((_K, n, 1), lambda i: (i, 0, 0)),
            pl.BlockSpec((_K, n, 4), lambda i: (i, 0, 0)),
            pl.BlockSpec((_K, n, 1), lambda i: (i, 0, 0)),
        ],
        out_shape=[
            jax.ShapeDtypeStruct((b, n, 1), jnp.int32),
            jax.ShapeDtypeStruct((b, n, 4), jnp.float32),
            jax.ShapeDtypeStruct((b, n, 1), jnp.float32),
        ],
        compiler_params=pltpu.CompilerParams(
            dimension_semantics=("arbitrary",)),
    )(*([pred_seq_logits] * _K), seq3, orig_f, size_f)
    return cls[..., 0], bbox, score[..., 0]


# speedup vs baseline: 1.0957x; 1.0178x over previous
import jax
import jax.numpy as jnp
from jax import lax
from jax.experimental import pallas as pl
from jax.experimental.pallas import tpu as pltpu

_BASE_VOCAB_SHIFT = 100
_COORD_VOCAB_SHIFT = 1000
_QUANT_BINS = 1000
_MAX_INPUT_SIZE = 1024.0
_K = 2


def _decode_one(x_all, k, seq_ref, orig_ref, size_ref,
                cls_ref, bbox_ref, score_ref):
    s, v = x_all.shape
    n = s // 5
    x = jnp.concatenate([x_all[5 * i + 4:5 * i + 5] for i in range(n)])
    m = jnp.max(x, axis=-1, keepdims=True)
    denom = jnp.sum(jnp.exp(x - m), axis=-1, keepdims=True)
    col = lax.broadcasted_iota(jnp.int32, (1, v), 1)
    inwin = (col >= _BASE_VOCAB_SHIFT) & (col < _COORD_VOCAB_SHIFT)
    xm = jnp.where(inwin, x, -jnp.inf)
    mw = jnp.max(xm, axis=-1, keepdims=True)
    idx = jnp.argmax(xm, axis=-1, keepdims=True)
    cls_ref[k] = jnp.maximum(idx - _BASE_VOCAB_SHIFT, 0)
    score_ref[k] = jnp.exp(mw - m) / denom

    sq = seq_ref[k]
    q = (sq - _COORD_VOCAB_SHIFT).astype(jnp.float32) / (_QUANT_BINS - 1)
    q = jnp.clip(q, 0.0, 1.0)
    sc = (_MAX_INPUT_SIZE / size_ref[k]) * orig_ref[k]
    bbox = jnp.concatenate(
        [q[:, 1:2], q[:, 0:1], q[:, 3:4], q[:, 2:3]], axis=1)
    scl4 = jnp.concatenate(
        [sc[:, 0:1], sc[:, 1:2], sc[:, 0:1], sc[:, 1:2]], axis=1)
    bbox_ref[k] = bbox * scl4


def _decode_body(*refs):
    logits_refs = refs[:_K]
    seq_ref, orig_ref, size_ref, cls_ref, bbox_ref, score_ref = refs[_K:]
    for k in range(_K):
        _decode_one(logits_refs[k][0], k, seq_ref, orig_ref, size_ref,
                    cls_ref, bbox_ref, score_ref)


def kernel(pred_seq_logits, pred_seq, orig_size, size, image_id):
    b, s, v = pred_seq_logits.shape
    n = s // 5
    seq3 = pred_seq.reshape(b, n, 5)
    orig_f = orig_size.astype(jnp.float32).reshape(b, 1, 2)
    size_f = size.astype(jnp.float32).reshape(b, 1, 2)

    def logits_spec(k):
        return pl.BlockSpec((1, s, v), lambda i, k=k: (_K * i + k, 0, 0))

    cls, bbox, score = pl.pallas_call(
        _decode_body,
        grid=(b // _K,),
        in_specs=[logits_spec(k) for k in range(_K)] + [
            pl.BlockSpec((_K, n, 5), lambda i: (i, 0, 0)),
            pl.BlockSpec((_K, 1, 2), lambda i: (i, 0, 0)),
            pl.BlockSpec((_K, 1, 2), lambda i: (i, 0, 0)),
        ],
        out_specs=[
            pl.BlockSpec((_K, n, 1), lambda i: (i, 0, 0)),
            pl.BlockSpec((_K, n, 4), lambda i: (i, 0, 0)),
            pl.BlockSpec((_K, n, 1), lambda i: (i, 0, 0)),
        ],
        out_shape=[
            jax.ShapeDtypeStruct((b, n, 1), jnp.int32),
            jax.ShapeDtypeStruct((b, n, 4), jnp.float32),
            jax.ShapeDtypeStruct((b, n, 1), jnp.float32),
        ],
        compiler_params=pltpu.CompilerParams(
            dimension_semantics=("arbitrary",)),
    )(*([pred_seq_logits] * _K), seq3, orig_f, size_f)
    return cls[..., 0], bbox, score[..., 0]


# speedup vs baseline: 1.1202x; 1.0224x over previous
import jax
import jax.numpy as jnp
from jax import lax
from jax.experimental import pallas as pl
from jax.experimental.pallas import tpu as pltpu

_BASE_VOCAB_SHIFT = 100
_COORD_VOCAB_SHIFT = 1000
_QUANT_BINS = 1000
_MAX_INPUT_SIZE = 1024.0
_K = 2


def _decode_one(x_all, k, seq_ref, orig_ref, size_ref,
                cls_ref, bbox_ref, score_ref):
    s, v = x_all.shape
    n = s // 5
    x = jnp.concatenate([x_all[5 * i + 4:5 * i + 5] for i in range(n)])
    m = jnp.max(x, axis=-1, keepdims=True)
    e = jnp.exp(x - m)
    denom = jnp.sum(e, axis=-1, keepdims=True)
    p = e / denom
    col = lax.broadcasted_iota(jnp.int32, (1, v), 1)
    inwin = (col >= _BASE_VOCAB_SHIFT) & (col < _COORD_VOCAB_SHIFT)
    pm = jnp.where(inwin, p, 0.0)
    idx = jnp.argmax(pm, axis=-1, keepdims=True)
    cls_ref[k] = jnp.maximum(idx - _BASE_VOCAB_SHIFT, 0)
    score_ref[k] = jnp.max(pm, axis=-1, keepdims=True)

    sq = seq_ref[k]
    q = (sq - _COORD_VOCAB_SHIFT).astype(jnp.float32) / (_QUANT_BINS - 1)
    q = jnp.clip(q, 0.0, 1.0)
    sc = (_MAX_INPUT_SIZE / size_ref[k]) * orig_ref[k]
    bbox = jnp.concatenate(
        [q[:, 1:2], q[:, 0:1], q[:, 3:4], q[:, 2:3]], axis=1)
    scl4 = jnp.concatenate(
        [sc[:, 0:1], sc[:, 1:2], sc[:, 0:1], sc[:, 1:2]], axis=1)
    bbox_ref[k] = bbox * scl4


def _decode_body(*refs):
    logits_refs = refs[:_K]
    seq_ref, orig_ref, size_ref, cls_ref, bbox_ref, score_ref = refs[_K:]
    for k in range(_K):
        _decode_one(logits_refs[k][0], k, seq_ref, orig_ref, size_ref,
                    cls_ref, bbox_ref, score_ref)


def kernel(pred_seq_logits, pred_seq, orig_size, size, image_id):
    b, s, v = pred_seq_logits.shape
    n = s // 5
    seq3 = pred_seq.reshape(b, n, 5)
    orig_f = orig_size.astype(jnp.float32).reshape(b, 1, 2)
    size_f = size.astype(jnp.float32).reshape(b, 1, 2)

    def logits_spec(k):
        return pl.BlockSpec((1, s, v), lambda i, k=k: (_K * i + k, 0, 0))

    cls, bbox, score = pl.pallas_call(
        _decode_body,
        grid=(b // _K,),
        in_specs=[logits_spec(k) for k in range(_K)] + [
            pl.BlockSpec((_K, n, 5), lambda i: (i, 0, 0)),
            pl.BlockSpec((_K, 1, 2), lambda i: (i, 0, 0)),
            pl.BlockSpec((_K, 1, 2), lambda i: (i, 0, 0)),
        ],
        out_specs=[
            pl.BlockSpec((_K, n, 1), lambda i: (i, 0, 0)),
            pl.BlockSpec((_K, n, 4), lambda i: (i, 0, 0)),
            pl.BlockSpec((_K, n, 1), lambda i: (i, 0, 0)),
        ],
        out_shape=[
            jax.ShapeDtypeStruct((b, n, 1), jnp.int32),
            jax.ShapeDtypeStruct((b, n, 4), jnp.float32),
            jax.ShapeDtypeStruct((b, n, 1), jnp.float32),
        ],
        compiler_params=pltpu.CompilerParams(
            dimension_semantics=("arbitrary",)),
    )(*([pred_seq_logits] * _K), seq3, orig_f, size_f)
    return cls[..., 0], bbox, score[..., 0]
